# Initial kernel scaffold; baseline (speedup 1.0000x reference)
import jax
import jax.numpy as jnp
from jax.experimental import pallas as pl


def kernel(atom_types, edge_index, bond_types, dist_bins, pos, atom_table, bond_table, dist_table):
    raise NotImplementedError("write your pallas kernel here")


# speedup vs baseline: 2.4594x; 2.4594x over previous
import jax
import jax.numpy as jnp
from jax import lax
from jax.experimental import pallas as pl
from jax.experimental.pallas import tpu as pltpu
from jax.experimental.pallas import tpu_sc as plsc

N_NODES = 50000
N_EDGES = 800000
EMB = 64
NC, NS = 2, 16
NW = NC * NS
L = 16

CH = 512
NFULL = N_EDGES // CH
TAIL_E = N_EDGES - NFULL * CH
TAIL_WID = NFULL % NW

NODE_FULL = N_NODES // CH
NODE_TAIL = N_NODES - NODE_FULL * CH
NODE_PAD = 50048

DEG_ROWS = 6256
DEG_GROUPS = DEG_ROWS // 16
HIST_PAD = 50048
DEG_SLICE = 3200


def _sc_body(atom1d, bins1d, bonds1d, src1d, dst1d, dstdeg2d, px, py, pz,
             atab, btab, dtab,
             node_out, dis_out, bond_out, deg_out, dirx_out, diry_out,
             dirz_out,
             idx_a, idx_b, idx_s, idx_d, rows_a, rows_b,
             pxd, pyd, pzd, pxs, pys, pzs, dx_b, dy_b, dz_b,
             onesb, dstb, zb, hist_sh, sem, sem2):
  cid = lax.axis_index("c")
  tid = lax.axis_index("s")
  wid = tid * NC + cid
  iota = lax.iota(jnp.int32, L)
  c0 = jnp.zeros((L,), jnp.int32)
  c1 = jnp.full((L,), 1, jnp.int32)
  c2 = jnp.full((L,), 2, jnp.int32)

  def echunk(c, S):
    K = S // 128
    b = c * CH
    pltpu.sync_copy(bins1d.at[pl.ds(b, S)], idx_a.at[pl.ds(0, S)])
    pltpu.sync_copy(bonds1d.at[pl.ds(b, S)], idx_b.at[pl.ds(0, S)])
    pltpu.sync_copy(src1d.at[pl.ds(b, S)], idx_s.at[pl.ds(0, S)])
    pltpu.sync_copy(dst1d.at[pl.ds(b, S)], idx_d.at[pl.ds(0, S)])
    cps = []
    for j in range(K):
      sl = pl.ds(j * 128, 128)
      cps.append(pltpu.async_copy(dtab.at[idx_a.at[sl]], rows_a.at[sl], sem))
      cps.append(pltpu.async_copy(btab.at[idx_b.at[sl]], rows_b.at[sl], sem))
      cps.append(pltpu.async_copy(px.at[idx_d.at[sl]], pxd.at[sl], sem2))
      cps.append(pltpu.async_copy(py.at[idx_d.at[sl]], pyd.at[sl], sem2))
      cps.append(pltpu.async_copy(pz.at[idx_d.at[sl]], pzd.at[sl], sem2))
      cps.append(pltpu.async_copy(px.at[idx_s.at[sl]], pxs.at[sl], sem2))
      cps.append(pltpu.async_copy(py.at[idx_s.at[sl]], pys.at[sl], sem2))
      cps.append(pltpu.async_copy(pz.at[idx_s.at[sl]], pzs.at[sl], sem2))
    for cp in cps:
      cp.wait()
    pltpu.sync_copy(rows_a.at[pl.ds(0, S)], dis_out.at[pl.ds(b, S)])
    pltpu.sync_copy(rows_b.at[pl.ds(0, S)], bond_out.at[pl.ds(b, S)])

    def gbody(g, carry):
      rows = g * L + iota
      sl16 = pl.ds(g * L, L)
      xd = pxd[sl16]
      yd = pyd[sl16]
      zd = pzd[sl16]
      xs = pxs[sl16]
      ys = pys[sl16]
      zs = pzs[sl16]
      dx = xd - xs
      dy = yd - ys
      dz = zd - zs
      s = dx * dx + dy * dy + dz * dz
      ib = lax.bitcast_convert_type(s, jnp.int32)
      y = lax.bitcast_convert_type(
          jnp.int32(0x5F3759DF) - lax.shift_right_logical(ib, 1), jnp.float32)
      half = s * jnp.float32(0.5)
      for _ in range(3):
        y = y * (jnp.float32(1.5) - half * y * y)
      nrm = s * y
      inv = jnp.float32(1.0) / (nrm + jnp.float32(1e-8))
      dx_b[sl16] = dx * inv
      dy_b[sl16] = dy * inv
      dz_b[sl16] = dz * inv
      return carry

    lax.fori_loop(0, S // L, gbody, 0)
    pltpu.sync_copy(dx_b.at[pl.ds(0, S)], dirx_out.at[pl.ds(b, S)])
    pltpu.sync_copy(dy_b.at[pl.ds(0, S)], diry_out.at[pl.ds(b, S)])
    pltpu.sync_copy(dz_b.at[pl.ds(0, S)], dirz_out.at[pl.ds(b, S)])

  @pl.when(cid == 0)
  def _degree():
    def zfill(i, carry):
      zb[pl.ds(i * L, L)] = jnp.zeros((L,), jnp.float32)
      return carry
    lax.fori_loop(0, DEG_SLICE // L, zfill, 0)
    off = tid * DEG_SLICE

    @pl.when(tid < NS - 1)
    def _():
      pltpu.sync_copy(zb, hist_sh.at[pl.ds(off, DEG_SLICE)])

    @pl.when(tid == NS - 1)
    def _():
      pltpu.sync_copy(zb.at[pl.ds(0, 2048)], hist_sh.at[pl.ds(off, 2048)])

    def ofill(i, carry):
      onesb[pl.ds(i * L, L)] = jnp.ones((L,), jnp.float32)
      return carry
    lax.fori_loop(0, 128 // L, ofill, 0)
    plsc.subcore_barrier()

    def dgroup(j, carry):
      g = tid + j * NS

      @pl.when(g < DEG_GROUPS)
      def _():
        pltpu.sync_copy(dstdeg2d.at[pl.ds(g * 16, 16)], dstb)
        for jj in range(16):
          pltpu.sync_copy(onesb, hist_sh.at[dstb.at[jj]], add=True)
      return carry

    lax.fori_loop(0, (DEG_GROUPS + NS - 1) // NS, dgroup, 0)
    plsc.subcore_barrier()

    @pl.when(tid < NS - 1)
    def _():
      pltpu.sync_copy(hist_sh.at[pl.ds(off, DEG_SLICE)],
                      deg_out.at[pl.ds(off, DEG_SLICE)])

    @pl.when(tid == NS - 1)
    def _():
      pltpu.sync_copy(hist_sh.at[pl.ds(off, 2048)],
                      deg_out.at[pl.ds(off, 2048)])

  @pl.when(cid == 1)
  def _nodes():
    def nchunk(j, carry):
      c = tid + j * NS

      @pl.when(c < NODE_FULL)
      def _():
        b = c * CH
        pltpu.sync_copy(atom1d.at[pl.ds(b, CH)], idx_a)
        cps = []
        for jj in range(CH // 128):
          sl = pl.ds(jj * 128, 128)
          cps.append(pltpu.async_copy(atab.at[idx_a.at[sl]],
                                      rows_a.at[sl], sem))
        for cp in cps:
          cp.wait()
        pltpu.sync_copy(rows_a, node_out.at[pl.ds(b, CH)])

      @pl.when(c == NODE_FULL)
      def _():
        b = NODE_FULL * CH
        pltpu.sync_copy(atom1d.at[pl.ds(b, 384)], idx_a.at[pl.ds(0, 384)])
        cps = []
        for jj in range(3):
          sl = pl.ds(jj * 128, 128)
          cps.append(pltpu.async_copy(atab.at[idx_a.at[sl]],
                                      rows_a.at[sl], sem))
        for cp in cps:
          cp.wait()
        pltpu.sync_copy(rows_a.at[pl.ds(0, NODE_TAIL)],
                        node_out.at[pl.ds(b, NODE_TAIL)])
      return carry

    lax.fori_loop(0, (NODE_FULL + NS) // NS + 1, nchunk, 0)

  def jloop(j, carry):
    c = wid + j * NW

    @pl.when(c < NFULL)
    def _():
      echunk(c, CH)
    return carry

  lax.fori_loop(0, (NFULL + NW - 1) // NW, jloop, 0)

  @pl.when(wid == TAIL_WID)
  def _():
    echunk(jnp.int32(NFULL), TAIL_E)


_sc_call = pl.kernel(
    _sc_body,
    out_type=[
        jax.ShapeDtypeStruct((N_NODES, EMB), jnp.float32),
        jax.ShapeDtypeStruct((N_EDGES, EMB), jnp.float32),
        jax.ShapeDtypeStruct((N_EDGES, EMB), jnp.float32),
        jax.ShapeDtypeStruct((HIST_PAD,), jnp.float32),
        jax.ShapeDtypeStruct((N_EDGES,), jnp.float32),
        jax.ShapeDtypeStruct((N_EDGES,), jnp.float32),
        jax.ShapeDtypeStruct((N_EDGES,), jnp.float32),
    ],
    mesh=plsc.VectorSubcoreMesh(core_axis_name="c", subcore_axis_name="s"),
    compiler_params=pltpu.CompilerParams(use_tc_tiling_on_sc=False),
    scratch_types=[
        pltpu.VMEM((CH,), jnp.int32),
        pltpu.VMEM((CH,), jnp.int32),
        pltpu.VMEM((CH,), jnp.int32),
        pltpu.VMEM((CH,), jnp.int32),
        pltpu.VMEM((CH, EMB), jnp.float32),
        pltpu.VMEM((CH, EMB), jnp.float32),
        pltpu.VMEM((CH,), jnp.float32),
        pltpu.VMEM((CH,), jnp.float32),
        pltpu.VMEM((CH,), jnp.float32),
        pltpu.VMEM((CH,), jnp.float32),
        pltpu.VMEM((CH,), jnp.float32),
        pltpu.VMEM((CH,), jnp.float32),
        pltpu.VMEM((CH,), jnp.float32),
        pltpu.VMEM((CH,), jnp.float32),
        pltpu.VMEM((CH,), jnp.float32),
        pltpu.VMEM((128,), jnp.float32),
        pltpu.VMEM((16, 128), jnp.int32),
        pltpu.VMEM((DEG_SLICE,), jnp.float32),
        pltpu.VMEM_SHARED((HIST_PAD,), jnp.float32),
        pltpu.SemaphoreType.DMA,
        pltpu.SemaphoreType.DMA,
    ],
)


def kernel(atom_types, edge_index, bond_types, dist_bins, pos,
           atom_table, bond_table, dist_table):
  atom1d = jnp.pad(atom_types.astype(jnp.int32), (0, NODE_PAD - N_NODES))
  bins1d = dist_bins.astype(jnp.int32)
  bonds1d = bond_types.astype(jnp.int32)
  src1d = edge_index[0].astype(jnp.int32)
  dst1d = edge_index[1].astype(jnp.int32)
  dstdeg2d = jnp.pad(dst1d, (0, DEG_ROWS * 128 - N_EDGES),
                     constant_values=N_NODES).reshape(DEG_ROWS, 128)
  px = pos[:, 0]
  py = pos[:, 1]
  pz = pos[:, 2]
  node_feat, edge_dis, edge_bond, degree, dirx, diry, dirz = _sc_call(
      atom1d, bins1d, bonds1d, src1d, dst1d, dstdeg2d, px, py, pz,
      atom_table, bond_table, dist_table)
  return (node_feat, edge_dis, edge_bond, degree[:N_NODES],
          jnp.stack((dirx, diry, dirz), axis=-1))
